# baseline (speedup 1.0000x reference)
import functools

import jax
import jax.numpy as jnp
from jax import lax
from jax.experimental import pallas as pl
from jax.experimental.pallas import tpu as pltpu
from jax.experimental.pallas import tpu_sc as plsc

B = 16384
D = 64
W = 2 * D
NE = 1000000
NR = 1000
NW = 32
BPW = B // NW
CH = 128
NCHUNK = BPW // CH
NROUND = 2
RCH = NCHUNK // NROUND
RROWS = RCH * CH
NT = 3 * NCHUNK
L = 16

TB = 8192
TBL = TB.bit_length() - 1
TGRID = (NE + 4 * TB - 1) // (4 * TB)
NP = TGRID * TB


def _fold_body(b0_ref, b1_ref, b2_ref, b3_ref, out_ref):
    z = jnp.concatenate(
        [b0_ref[...], b1_ref[...], b2_ref[...], b3_ref[...]], axis=0)
    u = z.T.astype(jnp.bfloat16)
    ui = lax.bitcast_convert_type(u, jnp.uint16).astype(jnp.int32)
    blocks = []
    for q in range(4):
        lo = ui[:, q * D:q * D + D // 2]
        hi = ui[:, q * D + D // 2:(q + 1) * D]
        blocks.append((hi << 16) | lo)
    out_ref[...] = jnp.concatenate(blocks, axis=1)


@jax.jit
def _fold_tc(entt):
    clamp = NE // TB

    def spec(k):
        return pl.BlockSpec(
            (D, TB), lambda i: (0, jnp.minimum(4 * i + k, clamp)))

    return pl.pallas_call(
        _fold_body,
        grid=(TGRID,),
        in_specs=[spec(0), spec(1), spec(2), spec(3)],
        out_specs=pl.BlockSpec((TB, W), lambda i: (i, 0)),
        out_shape=jax.ShapeDtypeStruct((NP, W), jnp.int32),
    )(entt, entt, entt, entt)


def _transe_body(idx_hbm, ent_hbm, rel_hbm, out_hbm,
                 idxraw, idxpair, hrows, rrows, trows, outv, sem):
    wid = lax.axis_index("s") * 2 + lax.axis_index("c")

    pltpu.sync_copy(idx_hbm.at[wid], idxraw)

    def pair_ent(m, carry):
        row = m // 8 + (m // (8 * NCHUNK)) * NCHUNK
        sl = pl.ds((m % 8) * L, L)
        v = idxraw[row, sl]
        idxpair[row, sl] = ((v >> (TBL + 2)) << TBL) | (v & (TB - 1))
        return carry

    lax.fori_loop(0, 2 * NCHUNK * 8, pair_ent, 0)

    def pair_rel(m, carry):
        row = NCHUNK + m // 8
        sl = pl.ds((m % 8) * L, L)
        idxpair[row, sl] = idxraw[row, sl] >> 2
        return carry

    lax.fori_loop(0, NCHUNK * 8, pair_rel, 0)

    lane = lax.iota(jnp.int32, L)

    def fire(c, buf):
        dst = pl.ds(buf * CH, CH)
        return [
            pltpu.async_copy(ent_hbm.at[idxpair.at[c]], hrows.at[dst], sem),
            pltpu.async_copy(
                rel_hbm.at[idxpair.at[NCHUNK + c]], rrows.at[dst], sem),
            pltpu.async_copy(
                ent_hbm.at[idxpair.at[2 * NCHUNK + c]], trows.at[dst], sem),
        ]

    def compute(c, buf):
        def group_body(g, carry):
            psl = pl.ds(g * L, L)
            hqv = (idxraw[c, psl] >> TBL) & 3
            rqv = idxraw[NCHUNK + c, psl] & 3
            tqv = (idxraw[2 * NCHUNK + c, psl] >> TBL) & 3
            svec = jnp.zeros((L,), jnp.float32)
            for k in range(L):
                i = buf * CH + g * L + k
                offh = hqv[k] * (D // 2)
                offr = rqv[k] * (D // 2)
                offt = tqv[k] * (D // 2)
                acc = jnp.zeros((L,), jnp.float32)
                hi_mask = jnp.full((L,), -65536, jnp.int32)
                for m in range(D // (2 * L)):
                    hw = hrows[i, pl.ds(offh + m * L, L)]
                    rw = rrows[i, pl.ds(offr + m * L, L)]
                    tw = trows[i, pl.ds(offt + m * L, L)]
                    ha = plsc.bitcast(hw << 16, jnp.float32)
                    hb = plsc.bitcast(hw & hi_mask, jnp.float32)
                    ra = plsc.bitcast(rw << 16, jnp.float32)
                    rb = plsc.bitcast(rw & hi_mask, jnp.float32)
                    ta = plsc.bitcast(tw << 16, jnp.float32)
                    tb = plsc.bitcast(tw & hi_mask, jnp.float32)
                    da = (ha + ra) - ta
                    db = (hb + rb) - tb
                    acc = acc + da * da + db * db
                svec = jnp.where(lane == k, jnp.sum(acc), svec)
            x = svec + 1e-12
            xi = plsc.bitcast(x, jnp.int32)
            yi = jnp.full((L,), 0x5F3759DF, jnp.int32) - (xi >> 1)
            y = plsc.bitcast(yi, jnp.float32)
            for _ in range(3):
                y = y * (1.5 - 0.5 * x * y * y)
            outv[pl.ds(c * CH + g * L, L)] = -(x * y)
            return carry

        lax.fori_loop(0, CH // L, group_body, 0)

    inflight = fire(0, 0)
    for c in range(NCHUNK):
        nxt = []
        if c + 1 < NCHUNK:
            nxt = fire(c + 1, (c + 1) % 2)
        for cp in inflight:
            cp.wait()
        compute(c, c % 2)
        inflight = nxt

    pltpu.sync_copy(outv, out_hbm.at[pl.ds(wid * BPW, BPW)])


@jax.jit
def _transe_sc(idx_all, ent2, rel2):
    mesh = plsc.VectorSubcoreMesh(core_axis_name="c", subcore_axis_name="s")
    return pl.kernel(
        _transe_body,
        mesh=mesh,
        compiler_params=pltpu.CompilerParams(
            needs_layout_passes=False, use_tc_tiling_on_sc=True),
        out_type=jax.ShapeDtypeStruct((B,), jnp.float32),
        scratch_types=[
            pltpu.VMEM((NT, CH), jnp.int32),
            pltpu.VMEM((NT, CH), jnp.int32),
            pltpu.VMEM((2 * CH, W), jnp.int32),
            pltpu.VMEM((2 * CH, W), jnp.int32),
            pltpu.VMEM((2 * CH, W), jnp.int32),
            pltpu.VMEM((BPW,), jnp.float32),
            pltpu.SemaphoreType.DMA,
        ],
    )(idx_all, ent2, rel2)


def kernel(heads, relations, tails, entity_embed, relation_embed):
    ent2 = _fold_tc(entity_embed.T)
    ru = jax.lax.bitcast_convert_type(
        relation_embed.astype(jnp.bfloat16), jnp.uint16).astype(jnp.int32)
    rel2 = ((ru[:, D // 2:] << 16) | ru[:, :D // 2]).reshape(NR // 4, W)
    h32 = heads.astype(jnp.int32).reshape(NW, NCHUNK, CH)
    r32 = relations.astype(jnp.int32).reshape(NW, NCHUNK, CH)
    t32 = tails.astype(jnp.int32).reshape(NW, NCHUNK, CH)
    idx_all = jnp.concatenate([h32, r32, t32], axis=1)
    return _transe_sc(idx_all, ent2, rel2)


# speedup vs baseline: 1.2609x; 1.2609x over previous
import functools

import jax
import jax.numpy as jnp
from jax import lax
from jax.experimental import pallas as pl
from jax.experimental.pallas import tpu as pltpu
from jax.experimental.pallas import tpu_sc as plsc

B = 16384
D = 64
W = 2 * D
NE = 1000000
NR = 1000
NW = 32
BPW = B // NW
CH = 128
NCHUNK = BPW // CH
NROUND = 2
RCH = NCHUNK // NROUND
RROWS = RCH * CH
NT = 3 * NCHUNK
L = 16

TB = 16384
TBL = TB.bit_length() - 1
TGRID = (NE + 2 * TB - 1) // (2 * TB)
NP = TGRID * TB


def _fold_body(lo_ref, hi_ref, out_ref):
    z = jnp.concatenate([lo_ref[...], hi_ref[...]], axis=0)
    out_ref[...] = z.T


@jax.jit
def _fold_tc(entt):
    return pl.pallas_call(
        _fold_body,
        grid=(TGRID,),
        in_specs=[
            pl.BlockSpec((D, TB), lambda i: (0, 2 * i)),
            pl.BlockSpec(
                (D, TB),
                lambda i: (0, jnp.minimum(2 * i + 1, NE // TB))),
        ],
        out_specs=pl.BlockSpec((TB, W), lambda i: (i, 0)),
        out_shape=jax.ShapeDtypeStruct((NP, W), jnp.float32),
    )(entt, entt)


def _transe_body(idx_hbm, ent_hbm, rel_hbm, out_hbm,
                 idxraw, idxpair, hrows, rrows, trows, outv, sem):
    wid = lax.axis_index("s") * 2 + lax.axis_index("c")

    pltpu.sync_copy(idx_hbm.at[wid], idxraw)

    def pair_ent(m, carry):
        row = m // 8 + (m // (8 * NCHUNK)) * NCHUNK
        sl = pl.ds((m % 8) * L, L)
        v = idxraw[row, sl]
        idxpair[row, sl] = ((v >> (TBL + 1)) << TBL) | (v & (TB - 1))
        return carry

    lax.fori_loop(0, 2 * NCHUNK * 8, pair_ent, 0)

    def pair_rel(m, carry):
        row = NCHUNK + m // 8
        sl = pl.ds((m % 8) * L, L)
        idxpair[row, sl] = idxraw[row, sl] >> 1
        return carry

    lax.fori_loop(0, NCHUNK * 8, pair_rel, 0)

    lane = lax.iota(jnp.int32, L)

    def fire(c, buf):
        dst = pl.ds(buf * CH, CH)
        return [
            pltpu.async_copy(ent_hbm.at[idxpair.at[c]], hrows.at[dst], sem),
            pltpu.async_copy(
                rel_hbm.at[idxpair.at[NCHUNK + c]], rrows.at[dst], sem),
            pltpu.async_copy(
                ent_hbm.at[idxpair.at[2 * NCHUNK + c]], trows.at[dst], sem),
        ]

    def compute(c, buf):
        def group_body(g, carry):
            psl = pl.ds(g * L, L)
            hqv = (idxraw[c, psl] >> TBL) & 1
            rqv = idxraw[NCHUNK + c, psl] & 1
            tqv = (idxraw[2 * NCHUNK + c, psl] >> TBL) & 1
            svec = jnp.zeros((L,), jnp.float32)
            for k in range(L):
                i = buf * CH + g * L + k
                offh = hqv[k] * D
                offr = rqv[k] * D
                offt = tqv[k] * D
                acc = jnp.zeros((L,), jnp.float32)
                for j in range(D // L):
                    h = hrows[i, pl.ds(offh + j * L, L)]
                    r = rrows[i, pl.ds(offr + j * L, L)]
                    t = trows[i, pl.ds(offt + j * L, L)]
                    d = (h + r) - t
                    acc = acc + d * d
                svec = jnp.where(lane == k, jnp.sum(acc), svec)
            x = svec + 1e-12
            xi = plsc.bitcast(x, jnp.int32)
            yi = jnp.full((L,), 0x5F3759DF, jnp.int32) - (xi >> 1)
            y = plsc.bitcast(yi, jnp.float32)
            for _ in range(3):
                y = y * (1.5 - 0.5 * x * y * y)
            outv[pl.ds(c * CH + g * L, L)] = -(x * y)
            return carry

        lax.fori_loop(0, CH // L, group_body, 0)

    inflight = fire(0, 0)
    for c in range(NCHUNK):
        nxt = []
        if c + 1 < NCHUNK:
            nxt = fire(c + 1, (c + 1) % 2)
        for cp in inflight:
            cp.wait()
        compute(c, c % 2)
        inflight = nxt

    pltpu.sync_copy(outv, out_hbm.at[pl.ds(wid * BPW, BPW)])


@jax.jit
def _transe_sc(idx_all, ent2, rel2):
    mesh = plsc.VectorSubcoreMesh(core_axis_name="c", subcore_axis_name="s")
    return pl.kernel(
        _transe_body,
        mesh=mesh,
        compiler_params=pltpu.CompilerParams(
            needs_layout_passes=False, use_tc_tiling_on_sc=True),
        out_type=jax.ShapeDtypeStruct((B,), jnp.float32),
        scratch_types=[
            pltpu.VMEM((NT, CH), jnp.int32),
            pltpu.VMEM((NT, CH), jnp.int32),
            pltpu.VMEM((2 * CH, W), jnp.float32),
            pltpu.VMEM((2 * CH, W), jnp.float32),
            pltpu.VMEM((2 * CH, W), jnp.float32),
            pltpu.VMEM((BPW,), jnp.float32),
            pltpu.SemaphoreType.DMA,
        ],
    )(idx_all, ent2, rel2)


def kernel(heads, relations, tails, entity_embed, relation_embed):
    ent2 = _fold_tc(entity_embed.T)
    rel2 = relation_embed.reshape(NR // 2, W)
    h32 = heads.astype(jnp.int32).reshape(NW, NCHUNK, CH)
    r32 = relations.astype(jnp.int32).reshape(NW, NCHUNK, CH)
    t32 = tails.astype(jnp.int32).reshape(NW, NCHUNK, CH)
    idx_all = jnp.concatenate([h32, r32, t32], axis=1)
    return _transe_sc(idx_all, ent2, rel2)
